# baseline (speedup 1.0000x reference)
import functools

import jax
import jax.numpy as jnp
from jax import lax
from jax.experimental import pallas as pl
from jax.experimental.pallas import tpu as pltpu
from jax.experimental.pallas import tpu_sc as plsc

NC = 2
NS = 16
L = 16
CHUNK = 128
SHIFT = 14

F32 = jnp.float32


def _mesh():
    return plsc.VectorSubcoreMesh(
        core_axis_name="c", subcore_axis_name="s", num_cores=NC, num_subcores=NS
    )


def _unpack_chunk(pk_v, j, slot, src_c, dst_c):
    mask = jnp.full((L,), (1 << SHIFT) - 1, jnp.int32)
    for v in range(CHUNK // L):
        p = pk_v[j, pl.ds(v * L, L)]
        if src_c is not None:
            src_c[slot, pl.ds(v * L, L)] = p >> SHIFT
        dst_c[slot, pl.ds(v * L, L)] = p & mask


def _make_deg_kernel(npad, nchp):
    rows = npad // NS

    @functools.partial(
        pl.kernel,
        out_type=jax.ShapeDtypeStruct((NC, npad), F32),
        mesh=_mesh(),
        scratch_types=[
            pltpu.VMEM((nchp, CHUNK), jnp.int32),
            pltpu.VMEM((1, CHUNK), jnp.int32),
            pltpu.VMEM((CHUNK,), F32),
            pltpu.VMEM((rows,), F32),
            pltpu.VMEM_SHARED((npad,), F32),
        ],
    )
    def deg_kernel(pk3, deg_out, pk_v, dst_c, ones_v, buf_v, acc):
        c = lax.axis_index("c")
        s = lax.axis_index("s")
        w = c * NS + s

        def fill_ones(i, _):
            ones_v[pl.ds(i * L, L)] = jnp.ones((L,), F32)
            return 0

        lax.fori_loop(0, CHUNK // L, fill_ones, 0)

        def fill_zero(i, _):
            buf_v[pl.ds(i * L, L)] = jnp.zeros((L,), F32)
            return 0

        lax.fori_loop(0, rows // L, fill_zero, 0)
        pltpu.sync_copy(buf_v, acc.at[pl.ds(s * rows, rows)])
        plsc.subcore_barrier()

        pltpu.sync_copy(pk3.at[w], pk_v)

        def step(j, _):
            _unpack_chunk(pk_v, j, 0, None, dst_c)
            pltpu.sync_copy(ones_v, acc.at[dst_c.at[0]], add=True)
            return 0

        lax.fori_loop(0, nchp, step, 0)
        plsc.subcore_barrier()

        pltpu.sync_copy(acc.at[pl.ds(s * rows, rows)], buf_v)
        pltpu.sync_copy(buf_v, deg_out.at[c, pl.ds(s * rows, rows)])

    return deg_kernel


def _make_agg_kernel(n, d, npad, nch):
    rows = npad // NS
    bb = 64
    passes = rows // bb
    assert nch % 2 == 0

    @functools.partial(
        pl.kernel,
        out_type=jax.ShapeDtypeStruct((NC, npad, d), F32),
        mesh=_mesh(),
        scratch_types=[
            pltpu.VMEM((nch + 2, CHUNK), jnp.int32),
            pltpu.VMEM((2, CHUNK), jnp.int32),
            pltpu.VMEM((2, CHUNK), jnp.int32),
            pltpu.VMEM((CHUNK, d), F32),
            pltpu.VMEM((CHUNK, d), F32),
            pltpu.VMEM_SHARED((npad, d), F32),
            pltpu.SemaphoreType.DMA,
            pltpu.SemaphoreType.DMA,
        ],
    )
    def agg_kernel(y_hbm, pk3, agg_out,
                   pk_v, src_c, dst_c, rows0, rows1, acc, g0, g1):
        c = lax.axis_index("c")
        s = lax.axis_index("s")
        w = c * NS + s
        bufs = (rows0, rows1)
        gsem = (g0, g1)

        def g_start(b):
            pltpu.async_copy(y_hbm.at[src_c.at[b]], bufs[b], gsem[b])

        def g_wait(b):
            pltpu.make_async_copy(y_hbm.at[src_c.at[b]], bufs[b], gsem[b]).wait()

        def fillz(i, _):
            for k in range(d // L):
                rows0[i, pl.ds(k * L, L)] = jnp.zeros((L,), F32)
            return 0

        lax.fori_loop(0, bb, fillz, 0)

        def zstep(p, _):
            pltpu.sync_copy(rows0.at[pl.ds(0, bb)],
                            acc.at[pl.ds(s * rows + p * bb, bb)])
            return 0

        lax.fori_loop(0, passes, zstep, 0)
        plsc.subcore_barrier()

        pltpu.sync_copy(pk3.at[w], pk_v)

        _unpack_chunk(pk_v, 0, 0, src_c, dst_c)
        g_start(0)
        _unpack_chunk(pk_v, 1, 1, src_c, dst_c)
        g_start(1)

        def pair(j2, _):
            for b in range(2):
                cur = 2 * j2 + b
                g_wait(b)
                pltpu.sync_copy(bufs[b], acc.at[dst_c.at[b]], add=True)
                _unpack_chunk(pk_v, cur + 2, b, src_c, dst_c)
                g_start(b)
            return 0

        lax.fori_loop(0, nch // 2, pair, 0)

        g_wait(0)
        g_wait(1)
        plsc.subcore_barrier()

        def ostep(p, _):
            pltpu.sync_copy(acc.at[pl.ds(s * rows + p * bb, bb)],
                            rows0.at[pl.ds(0, bb)])
            pltpu.sync_copy(rows0.at[pl.ds(0, bb)],
                            agg_out.at[c, pl.ds(s * rows + p * bb, bb)])
            return 0

        lax.fori_loop(0, passes, ostep, 0)

    return agg_kernel


def _tc1_body(deg_ref, x_ref, w_ref, dinv_ref, y_ref):
    n = x_ref.shape[0]
    deg = deg_ref[0, :n] + deg_ref[1, :n] + 1.0
    dinv = lax.rsqrt(deg)[:, None]
    dinv_ref[...] = dinv
    xw = jnp.dot(x_ref[...], w_ref[...], preferred_element_type=F32)
    y_ref[...] = xw * dinv


def _tc2_body(dinv_ref, aggp_ref, y1_ref, b1_ref, w_ref, y2_ref):
    n = y1_ref.shape[0]
    dinv = dinv_ref[...]
    agg = aggp_ref[0, :n, :] + aggp_ref[1, :n, :]
    h = jnp.maximum(dinv * (agg + y1_ref[...]) + b1_ref[...][None, :], 0.0)
    y2_ref[...] = jnp.dot(h, w_ref[...], preferred_element_type=F32) * dinv


def _tc3_body(dinv_ref, aggp_ref, y2_ref, b2_ref, out_ref):
    n = y2_ref.shape[0]
    agg = aggp_ref[0, :n, :] + aggp_ref[1, :n, :]
    out_ref[...] = dinv_ref[...] * (agg + y2_ref[...]) + b2_ref[...][None, :]


def kernel(x, edge_index, W1, b1, W2, b2):
    n, d = x.shape
    e = edge_index.shape[1]
    assert n < (1 << SHIFT)
    per = NC * NS * CHUNK
    nch = -(-e // per)
    if nch % 2 == 1:
        nch += 1
    ep = nch * per
    npad = -(-(n + 1) // (NS * L)) * (NS * L)

    packed = (edge_index[0] << SHIFT) | edge_index[1]
    pad = ep - e
    if pad > 0:
        packed = jnp.concatenate([packed, jnp.full((pad,), n, jnp.int32)])
    pk3 = jnp.concatenate(
        [packed.reshape(NC * NS, nch, CHUNK),
         jnp.full((NC * NS, 2, CHUNK), n, jnp.int32)], axis=1)

    deg_p = _make_deg_kernel(npad, nch + 2)(pk3)

    tc1 = pl.pallas_call(
        _tc1_body,
        out_shape=(
            jax.ShapeDtypeStruct((n, 1), F32),
            jax.ShapeDtypeStruct((n, d), F32),
        ),
    )
    dinv, y1 = tc1(deg_p, x, W1)

    agg_call = _make_agg_kernel(n, d, npad, nch)
    agg1_p = agg_call(y1, pk3)

    tc2 = pl.pallas_call(
        _tc2_body,
        out_shape=jax.ShapeDtypeStruct((n, d), F32),
    )
    y2 = tc2(dinv, agg1_p, y1, b1, W2)

    agg2_p = agg_call(y2, pk3)

    tc3 = pl.pallas_call(
        _tc3_body,
        out_shape=jax.ShapeDtypeStruct((n, d), F32),
    )
    return tc3(dinv, agg2_p, y2, b2)


# speedup vs baseline: 1.1924x; 1.1924x over previous
import functools

import jax
import jax.numpy as jnp
from jax import lax
from jax.experimental import pallas as pl
from jax.experimental.pallas import tpu as pltpu
from jax.experimental.pallas import tpu_sc as plsc

NC = 2
NS = 16
L = 16
CHUNK = 128
SHIFT = 14

F32 = jnp.float32


def _mesh():
    return plsc.VectorSubcoreMesh(
        core_axis_name="c", subcore_axis_name="s", num_cores=NC, num_subcores=NS
    )


def _unpack_chunk(pk_v, j, slot, src_c, dst_c):
    mask = jnp.full((L,), (1 << SHIFT) - 1, jnp.int32)
    for v in range(CHUNK // L):
        p = pk_v[j, pl.ds(v * L, L)]
        if src_c is not None:
            src_c[slot, pl.ds(v * L, L)] = p >> SHIFT
        dst_c[slot, pl.ds(v * L, L)] = p & mask


def _make_deg_kernel(npad, nchp):
    rows = npad // NS

    @functools.partial(
        pl.kernel,
        out_type=jax.ShapeDtypeStruct((NC, npad), F32),
        mesh=_mesh(),
        scratch_types=[
            pltpu.VMEM((nchp, CHUNK), jnp.int32),
            pltpu.VMEM((1, CHUNK), jnp.int32),
            pltpu.VMEM((CHUNK,), F32),
            pltpu.VMEM((rows,), F32),
            pltpu.VMEM_SHARED((npad,), F32),
        ],
    )
    def deg_kernel(pk3, deg_out, pk_v, dst_c, ones_v, buf_v, acc):
        c = lax.axis_index("c")
        s = lax.axis_index("s")
        w = c * NS + s

        def fill_ones(i, _):
            ones_v[pl.ds(i * L, L)] = jnp.ones((L,), F32)
            return 0

        lax.fori_loop(0, CHUNK // L, fill_ones, 0)

        def fill_zero(i, _):
            buf_v[pl.ds(i * L, L)] = jnp.zeros((L,), F32)
            return 0

        lax.fori_loop(0, rows // L, fill_zero, 0)
        pltpu.sync_copy(buf_v, acc.at[pl.ds(s * rows, rows)])
        plsc.subcore_barrier()

        pltpu.sync_copy(pk3.at[w], pk_v)

        def step(j, _):
            _unpack_chunk(pk_v, j, 0, None, dst_c)
            pltpu.sync_copy(ones_v, acc.at[dst_c.at[0]], add=True)
            return 0

        lax.fori_loop(0, nchp, step, 0)
        plsc.subcore_barrier()

        pltpu.sync_copy(acc.at[pl.ds(s * rows, rows)], buf_v)
        pltpu.sync_copy(buf_v, deg_out.at[c, pl.ds(s * rows, rows)])

    return deg_kernel


def _make_agg_kernel(n, d, npad, nch):
    rows = npad // NS
    bb = 64
    passes = rows // bb
    assert nch % 2 == 0

    @functools.partial(
        pl.kernel,
        out_type=jax.ShapeDtypeStruct((NC, npad, d), F32),
        mesh=_mesh(),
        scratch_types=[
            pltpu.VMEM((nch + 2, CHUNK), jnp.int32),
            pltpu.VMEM((2, CHUNK), jnp.int32),
            pltpu.VMEM((2, CHUNK), jnp.int32),
            pltpu.VMEM((CHUNK, d), F32),
            pltpu.VMEM((CHUNK, d), F32),
            pltpu.VMEM_SHARED((npad, d), F32),
            pltpu.SemaphoreType.DMA,
        ],
    )
    def agg_kernel(y_hbm, pk3, agg_out,
                   pk_v, src_c, dst_c, rows0, rows1, acc, gsem):
        c = lax.axis_index("c")
        s = lax.axis_index("s")
        w = c * NS + s
        bufs = (rows0, rows1)

        def fillz(i, _):
            for k in range(d // L):
                rows0[i, pl.ds(k * L, L)] = jnp.zeros((L,), F32)
            return 0

        lax.fori_loop(0, bb, fillz, 0)

        def zstep(p, _):
            pltpu.sync_copy(rows0.at[pl.ds(0, bb)],
                            acc.at[pl.ds(s * rows + p * bb, bb)])
            return 0

        lax.fori_loop(0, passes, zstep, 0)
        plsc.subcore_barrier()

        pltpu.sync_copy(pk3.at[w], pk_v)

        _unpack_chunk(pk_v, 0, 0, src_c, dst_c)
        pltpu.sync_copy(y_hbm.at[src_c.at[0]], rows0)

        def pair(j2, _):
            for b in range(2):
                cur = 2 * j2 + b
                nb = 1 - b
                _unpack_chunk(pk_v, cur + 1, nb, src_c, dst_c)
                desc = pltpu.async_copy(y_hbm.at[src_c.at[nb]], bufs[nb], gsem)
                pltpu.sync_copy(bufs[b], acc.at[dst_c.at[b]], add=True)
                desc.wait()
            return 0

        lax.fori_loop(0, nch // 2, pair, 0)
        plsc.subcore_barrier()

        def ostep(p, _):
            pltpu.sync_copy(acc.at[pl.ds(s * rows + p * bb, bb)],
                            rows0.at[pl.ds(0, bb)])
            pltpu.sync_copy(rows0.at[pl.ds(0, bb)],
                            agg_out.at[c, pl.ds(s * rows + p * bb, bb)])
            return 0

        lax.fori_loop(0, passes, ostep, 0)

    return agg_kernel


def _tc1_body(deg_ref, x_ref, w_ref, dinv_ref, y_ref):
    n = x_ref.shape[0]
    deg = deg_ref[0, :n] + deg_ref[1, :n] + 1.0
    dinv = lax.rsqrt(deg)[:, None]
    dinv_ref[...] = dinv
    xw = jnp.dot(x_ref[...], w_ref[...], preferred_element_type=F32)
    y_ref[...] = xw * dinv


def _tc2_body(dinv_ref, aggp_ref, y1_ref, b1_ref, w_ref, y2_ref):
    n = y1_ref.shape[0]
    dinv = dinv_ref[...]
    agg = aggp_ref[0, :n, :] + aggp_ref[1, :n, :]
    h = jnp.maximum(dinv * (agg + y1_ref[...]) + b1_ref[...][None, :], 0.0)
    y2_ref[...] = jnp.dot(h, w_ref[...], preferred_element_type=F32) * dinv


def _tc3_body(dinv_ref, aggp_ref, y2_ref, b2_ref, out_ref):
    n = y2_ref.shape[0]
    agg = aggp_ref[0, :n, :] + aggp_ref[1, :n, :]
    out_ref[...] = dinv_ref[...] * (agg + y2_ref[...]) + b2_ref[...][None, :]


def kernel(x, edge_index, W1, b1, W2, b2):
    n, d = x.shape
    e = edge_index.shape[1]
    assert n < (1 << SHIFT)
    per = NC * NS * CHUNK
    nch = -(-e // per)
    if nch % 2 == 1:
        nch += 1
    ep = nch * per
    npad = -(-(n + 1) // (NS * L)) * (NS * L)

    packed = (edge_index[0] << SHIFT) | edge_index[1]
    pad = ep - e
    if pad > 0:
        packed = jnp.concatenate([packed, jnp.full((pad,), n, jnp.int32)])
    pk3 = jnp.concatenate(
        [packed.reshape(NC * NS, nch, CHUNK),
         jnp.full((NC * NS, 2, CHUNK), n, jnp.int32)], axis=1)

    deg_p = _make_deg_kernel(npad, nch + 2)(pk3)

    tc1 = pl.pallas_call(
        _tc1_body,
        out_shape=(
            jax.ShapeDtypeStruct((n, 1), F32),
            jax.ShapeDtypeStruct((n, d), F32),
        ),
    )
    dinv, y1 = tc1(deg_p, x, W1)

    agg_call = _make_agg_kernel(n, d, npad, nch)
    agg1_p = agg_call(y1, pk3)

    tc2 = pl.pallas_call(
        _tc2_body,
        out_shape=jax.ShapeDtypeStruct((n, d), F32),
    )
    y2 = tc2(dinv, agg1_p, y1, b1, W2)

    agg2_p = agg_call(y2, pk3)

    tc3 = pl.pallas_call(
        _tc3_body,
        out_shape=jax.ShapeDtypeStruct((n, d), F32),
    )
    return tc3(dinv, agg2_p, y2, b2)


# speedup vs baseline: 1.4264x; 1.1962x over previous
import functools

import jax
import jax.numpy as jnp
from jax import lax
from jax.experimental import pallas as pl
from jax.experimental.pallas import tpu as pltpu
from jax.experimental.pallas import tpu_sc as plsc

NC = 2
NS = 16
L = 16
CHUNK = 128
SHIFT = 14

F32 = jnp.float32


def _mesh():
    return plsc.VectorSubcoreMesh(
        core_axis_name="c", subcore_axis_name="s", num_cores=NC, num_subcores=NS
    )


def _unpack_chunk(pk_v, j, slot, src_c, dst_c):
    mask = jnp.full((L,), (1 << SHIFT) - 1, jnp.int32)
    for v in range(CHUNK // L):
        p = pk_v[j, pl.ds(v * L, L)]
        if src_c is not None:
            src_c[slot, pl.ds(v * L, L)] = p >> SHIFT
        dst_c[slot, pl.ds(v * L, L)] = p & mask


def _make_deg_kernel(npad, nchp):
    rows = npad // NS

    @functools.partial(
        pl.kernel,
        out_type=jax.ShapeDtypeStruct((NC, npad), F32),
        mesh=_mesh(),
        scratch_types=[
            pltpu.VMEM((nchp, CHUNK), jnp.int32),
            pltpu.VMEM((1, CHUNK), jnp.int32),
            pltpu.VMEM((CHUNK,), F32),
            pltpu.VMEM((rows,), F32),
            pltpu.VMEM_SHARED((npad,), F32),
        ],
    )
    def deg_kernel(pk3, deg_out, pk_v, dst_c, ones_v, buf_v, acc):
        c = lax.axis_index("c")
        s = lax.axis_index("s")
        w = c * NS + s

        def fill_ones(i, _):
            ones_v[pl.ds(i * L, L)] = jnp.ones((L,), F32)
            return 0

        lax.fori_loop(0, CHUNK // L, fill_ones, 0)

        def fill_zero(i, _):
            buf_v[pl.ds(i * L, L)] = jnp.zeros((L,), F32)
            return 0

        lax.fori_loop(0, rows // L, fill_zero, 0)
        pltpu.sync_copy(buf_v, acc.at[pl.ds(s * rows, rows)])
        plsc.subcore_barrier()

        pltpu.sync_copy(pk3.at[w], pk_v)

        def step(j, _):
            _unpack_chunk(pk_v, j, 0, None, dst_c)
            pltpu.sync_copy(ones_v, acc.at[dst_c.at[0]], add=True)
            return 0

        lax.fori_loop(0, nchp, step, 0)
        plsc.subcore_barrier()

        pltpu.sync_copy(acc.at[pl.ds(s * rows, rows)], buf_v)
        pltpu.sync_copy(buf_v, deg_out.at[c, pl.ds(s * rows, rows)])

    return deg_kernel


def _make_agg_kernel(n, d, npad, nch):
    rows = npad // NS
    bb = 64
    passes = rows // bb
    assert nch % 2 == 0

    @functools.partial(
        pl.kernel,
        out_type=jax.ShapeDtypeStruct((NC, npad, d), F32),
        mesh=_mesh(),
        scratch_types=[
            pltpu.VMEM((nch + 2, CHUNK), jnp.int32),
            pltpu.VMEM((2, CHUNK), jnp.int32),
            pltpu.VMEM((2, CHUNK), jnp.int32),
            pltpu.VMEM((CHUNK, d), F32),
            pltpu.VMEM((CHUNK, d), F32),
            pltpu.VMEM_SHARED((npad, d), F32),
            pltpu.SemaphoreType.DMA,
        ],
    )
    def agg_kernel(y_hbm, pk3, agg_out,
                   pk_v, src_c, dst_c, rows0, rows1, acc, gsem):
        c = lax.axis_index("c")
        s = lax.axis_index("s")
        w = c * NS + s
        bufs = (rows0, rows1)

        def fillz(i, _):
            for k in range(d // L):
                rows0[i, pl.ds(k * L, L)] = jnp.zeros((L,), F32)
            return 0

        lax.fori_loop(0, bb, fillz, 0)

        def zstep(p, _):
            pltpu.sync_copy(rows0.at[pl.ds(0, bb)],
                            acc.at[pl.ds(s * rows + p * bb, bb)])
            return 0

        lax.fori_loop(0, passes, zstep, 0)
        plsc.subcore_barrier()

        pltpu.sync_copy(pk3.at[w], pk_v)

        def pair(j2, _):
            for b in range(2):
                cur = 2 * j2 + b
                _unpack_chunk(pk_v, cur, b, src_c, dst_c)
                pltpu.sync_copy(y_hbm.at[src_c.at[b]], bufs[b])
                pltpu.sync_copy(bufs[b], acc.at[dst_c.at[b]], add=True)
            return 0

        lax.fori_loop(0, nch // 2, pair, 0)
        plsc.subcore_barrier()

        def ostep(p, _):
            pltpu.sync_copy(acc.at[pl.ds(s * rows + p * bb, bb)],
                            rows0.at[pl.ds(0, bb)])
            pltpu.sync_copy(rows0.at[pl.ds(0, bb)],
                            agg_out.at[c, pl.ds(s * rows + p * bb, bb)])
            return 0

        lax.fori_loop(0, passes, ostep, 0)

    return agg_kernel


def _tc1_body(deg_ref, x_ref, w_ref, dinv_ref, y_ref):
    n = x_ref.shape[0]
    deg = deg_ref[0, :n] + deg_ref[1, :n] + 1.0
    dinv = lax.rsqrt(deg)[:, None]
    dinv_ref[...] = dinv
    xw = jnp.dot(x_ref[...], w_ref[...], preferred_element_type=F32)
    y_ref[...] = xw * dinv


def _tc2_body(dinv_ref, aggp_ref, y1_ref, b1_ref, w_ref, y2_ref):
    n = y1_ref.shape[0]
    dinv = dinv_ref[...]
    agg = aggp_ref[0, :n, :] + aggp_ref[1, :n, :]
    h = jnp.maximum(dinv * (agg + y1_ref[...]) + b1_ref[...][None, :], 0.0)
    y2_ref[...] = jnp.dot(h, w_ref[...], preferred_element_type=F32) * dinv


def _tc3_body(dinv_ref, aggp_ref, y2_ref, b2_ref, out_ref):
    n = y2_ref.shape[0]
    agg = aggp_ref[0, :n, :] + aggp_ref[1, :n, :]
    out_ref[...] = dinv_ref[...] * (agg + y2_ref[...]) + b2_ref[...][None, :]


def kernel(x, edge_index, W1, b1, W2, b2):
    n, d = x.shape
    e = edge_index.shape[1]
    assert n < (1 << SHIFT)
    per = NC * NS * CHUNK
    nch = -(-e // per)
    if nch % 2 == 1:
        nch += 1
    ep = nch * per
    npad = -(-(n + 1) // (NS * L)) * (NS * L)

    packed = (edge_index[0] << SHIFT) | edge_index[1]
    pad = ep - e
    if pad > 0:
        packed = jnp.concatenate([packed, jnp.full((pad,), n, jnp.int32)])
    pk3 = jnp.concatenate(
        [packed.reshape(NC * NS, nch, CHUNK),
         jnp.full((NC * NS, 2, CHUNK), n, jnp.int32)], axis=1)

    deg_p = _make_deg_kernel(npad, nch + 2)(pk3)

    tc1 = pl.pallas_call(
        _tc1_body,
        out_shape=(
            jax.ShapeDtypeStruct((n, 1), F32),
            jax.ShapeDtypeStruct((n, d), F32),
        ),
    )
    dinv, y1 = tc1(deg_p, x, W1)

    agg_call = _make_agg_kernel(n, d, npad, nch)
    agg1_p = agg_call(y1, pk3)

    tc2 = pl.pallas_call(
        _tc2_body,
        out_shape=jax.ShapeDtypeStruct((n, d), F32),
    )
    y2 = tc2(dinv, agg1_p, y1, b1, W2)

    agg2_p = agg_call(y2, pk3)

    tc3 = pl.pallas_call(
        _tc3_body,
        out_shape=jax.ShapeDtypeStruct((n, d), F32),
    )
    return tc3(dinv, agg2_p, y2, b2)


# speedup vs baseline: 2.1861x; 1.5326x over previous
import functools

import jax
import jax.numpy as jnp
from jax import lax
from jax.experimental import pallas as pl
from jax.experimental.pallas import tpu as pltpu
from jax.experimental.pallas import tpu_sc as plsc

NC = 2
NS = 16
L = 16
CHUNK = 128

F32 = jnp.float32


def _mesh():
    return plsc.VectorSubcoreMesh(
        core_axis_name="c", subcore_axis_name="s", num_cores=NC, num_subcores=NS
    )


def _make_deg_kernel(npad, nch):
    rows = npad // NS

    @functools.partial(
        pl.kernel,
        out_type=jax.ShapeDtypeStruct((NC, npad), F32),
        mesh=_mesh(),
        scratch_types=[
            pltpu.VMEM((nch, CHUNK), jnp.int32),
            pltpu.VMEM((CHUNK,), F32),
            pltpu.VMEM((rows,), F32),
            pltpu.VMEM_SHARED((npad,), F32),
        ],
    )
    def deg_kernel(dst3, deg_out, idx_v, ones_v, buf_v, acc):
        c = lax.axis_index("c")
        s = lax.axis_index("s")
        w = c * NS + s

        def fill_ones(i, _):
            ones_v[pl.ds(i * L, L)] = jnp.ones((L,), F32)
            return 0

        lax.fori_loop(0, CHUNK // L, fill_ones, 0)

        def fill_zero(i, _):
            buf_v[pl.ds(i * L, L)] = jnp.zeros((L,), F32)
            return 0

        lax.fori_loop(0, rows // L, fill_zero, 0)
        pltpu.sync_copy(buf_v, acc.at[pl.ds(s * rows, rows)])
        plsc.subcore_barrier()

        pltpu.sync_copy(dst3.at[w], idx_v)

        def step(j, _):
            pltpu.sync_copy(ones_v, acc.at[idx_v.at[j]], add=True)
            return 0

        lax.fori_loop(0, nch, step, 0)
        plsc.subcore_barrier()

        pltpu.sync_copy(acc.at[pl.ds(s * rows, rows)],
                        deg_out.at[c, pl.ds(s * rows, rows)])

    return deg_kernel


def _make_agg_kernel(n, d, npad, nch):
    rows = npad // NS
    bb = 64
    passes = rows // bb

    @functools.partial(
        pl.kernel,
        out_type=jax.ShapeDtypeStruct((NC, npad, d), F32),
        mesh=_mesh(),
        scratch_types=[
            pltpu.VMEM((nch, CHUNK), jnp.int32),
            pltpu.VMEM((nch, CHUNK), jnp.int32),
            pltpu.VMEM((CHUNK, d), F32),
            pltpu.VMEM((bb, d), F32),
            pltpu.VMEM_SHARED((npad, d), F32),
        ],
    )
    def agg_kernel(y_hbm, src3, dst3, agg_out, src_v, dst_v, rows_v, buf_v, acc):
        c = lax.axis_index("c")
        s = lax.axis_index("s")
        w = c * NS + s

        def fillz(i, _):
            for k in range(d // L):
                buf_v[i, pl.ds(k * L, L)] = jnp.zeros((L,), F32)
            return 0

        lax.fori_loop(0, bb, fillz, 0)

        def zstep(p, _):
            pltpu.sync_copy(buf_v, acc.at[pl.ds(s * rows + p * bb, bb)])
            return 0

        lax.fori_loop(0, passes, zstep, 0)
        plsc.subcore_barrier()

        pltpu.sync_copy(src3.at[w], src_v)
        pltpu.sync_copy(dst3.at[w], dst_v)

        def step(j, _):
            pltpu.sync_copy(y_hbm.at[src_v.at[j]], rows_v)
            pltpu.sync_copy(rows_v, acc.at[dst_v.at[j]], add=True)
            return 0

        lax.fori_loop(0, nch, step, 0)
        plsc.subcore_barrier()

        pltpu.sync_copy(acc.at[pl.ds(s * rows, rows)],
                        agg_out.at[c, pl.ds(s * rows, rows)])

    return agg_kernel


def _tc1_body(deg_ref, x_ref, w_ref, dinv_ref, y_ref):
    n = x_ref.shape[0]
    deg = deg_ref[0, :n] + deg_ref[1, :n] + 1.0
    dinv = lax.rsqrt(deg)[:, None]
    dinv_ref[...] = dinv
    xw = jnp.dot(x_ref[...], w_ref[...], preferred_element_type=F32)
    y_ref[...] = xw * dinv


def _tc2_body(dinv_ref, aggp_ref, y1_ref, b1_ref, w_ref, y2_ref):
    n = y1_ref.shape[0]
    dinv = dinv_ref[...]
    agg = aggp_ref[0, :n, :] + aggp_ref[1, :n, :]
    h = jnp.maximum(dinv * (agg + y1_ref[...]) + b1_ref[...][None, :], 0.0)
    y2_ref[...] = jnp.dot(h, w_ref[...], preferred_element_type=F32) * dinv


def _tc3_body(dinv_ref, aggp_ref, y2_ref, b2_ref, out_ref):
    n = y2_ref.shape[0]
    agg = aggp_ref[0, :n, :] + aggp_ref[1, :n, :]
    out_ref[...] = dinv_ref[...] * (agg + y2_ref[...]) + b2_ref[...][None, :]


def kernel(x, edge_index, W1, b1, W2, b2):
    n, d = x.shape
    e = edge_index.shape[1]
    per = NC * NS * CHUNK
    nch = -(-e // per)
    ep = nch * per
    npad = -(-(n + 1) // (NS * L)) * (NS * L)

    src = edge_index[0]
    dst = edge_index[1]
    pad = ep - e
    if pad > 0:
        src = jnp.concatenate([src, jnp.zeros((pad,), jnp.int32)])
        dst = jnp.concatenate([dst, jnp.full((pad,), n, jnp.int32)])
    src3 = src.reshape(NC * NS, nch, CHUNK)
    dst3 = dst.reshape(NC * NS, nch, CHUNK)

    deg_p = _make_deg_kernel(npad, nch)(dst3)

    tc1 = pl.pallas_call(
        _tc1_body,
        out_shape=(
            jax.ShapeDtypeStruct((n, 1), F32),
            jax.ShapeDtypeStruct((n, d), F32),
        ),
    )
    dinv, y1 = tc1(deg_p, x, W1)

    agg_call = _make_agg_kernel(n, d, npad, nch)
    agg1_p = agg_call(y1, src3, dst3)

    tc2 = pl.pallas_call(
        _tc2_body,
        out_shape=jax.ShapeDtypeStruct((n, d), F32),
    )
    y2 = tc2(dinv, agg1_p, y1, b1, W2)

    agg2_p = agg_call(y2, src3, dst3)

    tc3 = pl.pallas_call(
        _tc3_body,
        out_shape=jax.ShapeDtypeStruct((n, d), F32),
    )
    return tc3(dinv, agg2_p, y2, b2)
